# Initial kernel scaffold; baseline (speedup 1.0000x reference)
import jax
import jax.numpy as jnp
from jax.experimental import pallas as pl


def kernel(query, query_pos, context, context_pos, rff_B, W1, b1, W2, b2, Wq, bq, Wk, bk, Wv, bv, Wo, bo):
    raise NotImplementedError("write your pallas kernel here")


# speedup vs baseline: 10.5758x; 10.5758x over previous
import functools

import jax
import jax.numpy as jnp
import numpy as np
from jax import lax
from jax.experimental import pallas as pl
from jax.experimental.pallas import tpu as pltpu
from jax.experimental.pallas import tpu_sc as plsc

E = 512
NUM_HEADS = 8
HD = E // NUM_HEADS
KNN = 16
B = 2
NQ = 2048
NC = 8192
NQT = B * NQ
NCT = B * NC
NCHUNK = 512
CHW = NC // NCHUNK

SC_CORES = 2
SC_SUBCORES = 16
SC_WORKERS = SC_CORES * SC_SUBCORES
QPW = NQT // SC_WORKERS


def _mlp_encode(pos, rff, W1, b1, W2, b2):
    proj = 2.0 * np.pi * jnp.dot(pos, rff, preferred_element_type=jnp.float32)
    feats = jnp.concatenate([jnp.cos(proj), jnp.sin(proj)], axis=-1)
    h = jax.nn.gelu(jnp.dot(feats, W1, preferred_element_type=jnp.float32) + b1)
    return jnp.dot(h, W2, preferred_element_type=jnp.float32) + b2


def _ctx_encode_kernel(pos_ref, rff_ref, W1_ref, b1_ref, W2_ref, b2_ref,
                       Wk_ref, bk_ref, Wv_ref, bv_ref, kc_ref, vc_ref):
    enc = _mlp_encode(pos_ref[...], rff_ref[...], W1_ref[...], b1_ref[...],
                      W2_ref[...], b2_ref[...])
    kc_ref[...] = jnp.dot(enc, Wk_ref[...], preferred_element_type=jnp.float32) + bk_ref[...]
    vc_ref[...] = jnp.dot(enc, Wv_ref[...], preferred_element_type=jnp.float32) + bv_ref[...]


def _qry_encode_kernel(pos_ref, rff_ref, W1_ref, b1_ref, W2_ref, b2_ref,
                       Wq_ref, bq_ref, q_ref):
    enc = _mlp_encode(pos_ref[...], rff_ref[...], W1_ref[...], b1_ref[...],
                      W2_ref[...], b2_ref[...])
    q_ref[...] = jnp.dot(enc, Wq_ref[...], preferred_element_type=jnp.float32) + bq_ref[...]


def _encode_project(qpos_pad, cpos_pad, rff_pad, W1, b1, W2, b2, Wq, bq, Wk, bk, Wv, bv):
    CB = 1024
    kc, vc = pl.pallas_call(
        _ctx_encode_kernel,
        grid=(NCT // CB,),
        in_specs=[
            pl.BlockSpec((CB, 8), lambda i: (i, 0)),
            pl.BlockSpec((8, E // 2), lambda i: (0, 0)),
            pl.BlockSpec((E, E), lambda i: (0, 0)),
            pl.BlockSpec((1, E), lambda i: (0, 0)),
            pl.BlockSpec((E, E), lambda i: (0, 0)),
            pl.BlockSpec((1, E), lambda i: (0, 0)),
            pl.BlockSpec((E, E), lambda i: (0, 0)),
            pl.BlockSpec((1, E), lambda i: (0, 0)),
            pl.BlockSpec((E, E), lambda i: (0, 0)),
            pl.BlockSpec((1, E), lambda i: (0, 0)),
        ],
        out_specs=[
            pl.BlockSpec((CB, E), lambda i: (i, 0)),
            pl.BlockSpec((CB, E), lambda i: (i, 0)),
        ],
        out_shape=[
            jax.ShapeDtypeStruct((NCT, E), jnp.float32),
            jax.ShapeDtypeStruct((NCT, E), jnp.float32),
        ],
    )(cpos_pad, rff_pad, W1, b1, W2, b2, Wk, bk, Wv, bv)

    QB = 1024
    qp = pl.pallas_call(
        _qry_encode_kernel,
        grid=(NQT // QB,),
        in_specs=[
            pl.BlockSpec((QB, 8), lambda i: (i, 0)),
            pl.BlockSpec((8, E // 2), lambda i: (0, 0)),
            pl.BlockSpec((E, E), lambda i: (0, 0)),
            pl.BlockSpec((1, E), lambda i: (0, 0)),
            pl.BlockSpec((E, E), lambda i: (0, 0)),
            pl.BlockSpec((1, E), lambda i: (0, 0)),
            pl.BlockSpec((E, E), lambda i: (0, 0)),
            pl.BlockSpec((1, E), lambda i: (0, 0)),
        ],
        out_specs=pl.BlockSpec((QB, E), lambda i: (i, 0)),
        out_shape=jax.ShapeDtypeStruct((NQT, E), jnp.float32),
    )(qpos_pad, rff_pad, W1, b1, W2, b2, Wq, bq)
    return qp, kc, vc


def _chunkmin_kernel(qpos_ref, cposT_ref, m_ref, d_ref):
    qp = qpos_ref[0]
    qb = qp.shape[0]
    qn2 = jnp.sum(qp * qp, axis=1, keepdims=True)

    def cm_body(t, m):
        cpt = cposT_ref[0, :, pl.ds(t * NCHUNK, NCHUNK)]
        cn2t = jnp.sum(cpt * cpt, axis=0)
        e = jnp.dot(qp, cpt, preferred_element_type=jnp.float32)
        dt = (qn2 + cn2t[None, :]) - 2.0 * e
        d_ref[0, :, pl.ds(t * NCHUNK, NCHUNK)] = dt
        return jnp.minimum(m, dt)

    m_ref[0] = lax.fori_loop(0, CHW, cm_body,
                             jnp.full((qb, NCHUNK), jnp.inf, jnp.float32))


def _select_kernel(m_ref, j_ref):
    m0 = m_ref[0]
    qb = m0.shape[0]
    lane = lax.broadcasted_iota(jnp.int32, (qb, NCHUNK), 1)
    lane16 = lax.broadcasted_iota(jnp.int32, (qb, KNN), 1)

    def sel_body(i, carry):
        m, jacc = carry
        mn = jnp.min(m, axis=1, keepdims=True)
        idx = jnp.min(jnp.where(m <= mn, lane, NCHUNK), axis=1, keepdims=True)
        jacc = jnp.where(lane16 == i, idx, jacc)
        m = jnp.where(lane == idx, jnp.inf, m)
        return m, jacc

    _, jacc = lax.fori_loop(0, KNN, sel_body,
                            (m0, jnp.zeros((qb, KNN), jnp.int32)))
    j_ref[0] = jacc


def _knn_chunks(qpos_pad, cposT):
    QB = 128
    m, dfull = pl.pallas_call(
        _chunkmin_kernel,
        grid=(B, NQ // QB),
        in_specs=[
            pl.BlockSpec((1, QB, 8), lambda b, i: (b, i, 0)),
            pl.BlockSpec((1, 8, NC), lambda b, i: (b, 0, 0)),
        ],
        out_specs=[
            pl.BlockSpec((1, QB, NCHUNK), lambda b, i: (b, i, 0)),
            pl.BlockSpec((1, QB, NC), lambda b, i: (b, i, 0)),
        ],
        out_shape=[
            jax.ShapeDtypeStruct((B, NQ, NCHUNK), jnp.float32),
            jax.ShapeDtypeStruct((B, NQ, NC), jnp.float32),
        ],
    )(qpos_pad, cposT)
    j = pl.pallas_call(
        _select_kernel,
        grid=(B, NQ // QB),
        in_specs=[pl.BlockSpec((1, QB, NCHUNK), lambda b, i: (b, i, 0))],
        out_specs=pl.BlockSpec((1, QB, KNN), lambda b, i: (b, i, 0)),
        out_shape=jax.ShapeDtypeStruct((B, NQ, KNN), jnp.int32),
    )(m)
    return j, dfull


def _sc_body(j_hbm, d_hbm, kc_hbm, vc_hbm,
             kg_hbm, vg_hbm,
             drow_v, j_v, krows, vrows, idx_v, sidx_v,
             sem_d, sem_k, sem_v):
    cid = lax.axis_index("c")
    sid = lax.axis_index("s")
    wid = sid * SC_CORES + cid
    b = wid // (SC_WORKERS // B)

    pltpu.sync_copy(j_hbm.at[pl.ds(wid * (QPW * KNN), QPW * KNN)], j_v)

    def body(ql, carry):
        q = wid * QPW + ql
        pltpu.async_copy(d_hbm.at[q], drow_v, sem_d).wait()
        jvec = j_v[pl.ds(ql * KNN, KNN)]
        rk = rv = None
        for t in range(CHW):
            cand = jvec + jnp.int32(t * NCHUNK)
            d2 = plsc.load_gather(drow_v, [cand])
            sk, sv = plsc.sort_key_val(d2, cand)
            if t == 0:
                rk, rv = sk, sv
            else:
                rsk = lax.rev(sk, (0,))
                rsv = lax.rev(sv, (0,))
                take = rk <= rsk
                mk = jnp.where(take, rk, rsk)
                mv = jnp.where(take, rv, rsv)
                rk, rv = plsc.sort_key_val(mk, mv)
        idx_v[...] = rv + b * NC
        sidx_v[...] = lax.iota(jnp.int32, 16) * NQT + q
        pltpu.async_copy(kc_hbm.at[idx_v], krows, sem_k).wait()
        pltpu.async_copy(vc_hbm.at[idx_v], vrows, sem_v).wait()
        pltpu.async_copy(krows, kg_hbm.at[sidx_v], sem_k).wait()
        pltpu.async_copy(vrows, vg_hbm.at[sidx_v], sem_v).wait()
        return carry

    lax.fori_loop(0, QPW, body, 0)


def _sc_refine_gather(j_flat, dflat, kc, vc):
    mesh = plsc.VectorSubcoreMesh(core_axis_name="c", subcore_axis_name="s",
                                  num_cores=SC_CORES, num_subcores=SC_SUBCORES)
    f = pl.kernel(
        _sc_body,
        compiler_params=pltpu.CompilerParams(needs_layout_passes=False),
        out_type=[
            jax.ShapeDtypeStruct((NQT * KNN, E), jnp.float32),
            jax.ShapeDtypeStruct((NQT * KNN, E), jnp.float32),
        ],
        mesh=mesh,
        scratch_types=[
            pltpu.VMEM((NC,), jnp.float32),
            pltpu.VMEM((QPW * KNN,), jnp.int32),
            pltpu.VMEM((KNN, E), jnp.float32),
            pltpu.VMEM((KNN, E), jnp.float32),
            pltpu.VMEM((KNN,), jnp.int32),
            pltpu.VMEM((KNN,), jnp.int32),
            pltpu.SemaphoreType.DMA,
            pltpu.SemaphoreType.DMA,
            pltpu.SemaphoreType.DMA,
        ],
    )
    return f(j_flat, dflat, kc, vc)


def _attn_kernel(q_ref, kg_ref, vg_ref, wseg_ref, wsegT_ref, Wo_ref, bo_ref, o_ref):
    q = q_ref[...]
    wseg = wseg_ref[...]
    wsegT = wsegT_ref[...]
    scale = 1.0 / np.sqrt(HD)
    sks = []
    m = None
    for k in range(KNN):
        sk = jnp.dot(q * kg_ref[k], wseg,
                     preferred_element_type=jnp.float32) * scale
        sks.append(sk)
        m = sk if k == 0 else jnp.maximum(m, sk)
    es = [jnp.exp(sk - m) for sk in sks]
    den = es[0]
    for e in es[1:]:
        den = den + e
    inv = 1.0 / den
    acc = None
    for k in range(KNN):
        w = jnp.dot(es[k], wsegT, preferred_element_type=jnp.float32)
        t = w * vg_ref[k]
        acc = t if k == 0 else acc + t
    o = acc * jnp.dot(inv, wsegT, preferred_element_type=jnp.float32)
    o_ref[...] = jnp.dot(o, Wo_ref[...], preferred_element_type=jnp.float32) + bo_ref[...]


def _attention(qp, kg, vg, wseg, Wo, bo):
    QB = 128
    return pl.pallas_call(
        _attn_kernel,
        grid=(NQT // QB,),
        in_specs=[
            pl.BlockSpec((QB, E), lambda i: (i, 0)),
            pl.BlockSpec((KNN, QB, E), lambda i: (0, i, 0)),
            pl.BlockSpec((KNN, QB, E), lambda i: (0, i, 0)),
            pl.BlockSpec((E, NUM_HEADS), lambda i: (0, 0)),
            pl.BlockSpec((NUM_HEADS, E), lambda i: (0, 0)),
            pl.BlockSpec((E, E), lambda i: (0, 0)),
            pl.BlockSpec((1, E), lambda i: (0, 0)),
        ],
        out_specs=pl.BlockSpec((QB, E), lambda i: (i, 0)),
        out_shape=jax.ShapeDtypeStruct((NQT, E), jnp.float32),
    )(qp, kg.reshape(KNN, NQT, E), vg.reshape(KNN, NQT, E), wseg, wseg.T, Wo, bo)


def kernel(query, query_pos, context, context_pos, rff_B, W1, b1, W2, b2,
           Wq, bq, Wk, bk, Wv, bv, Wo, bo):
    pad = [(0, 0), (0, 0), (0, 5)]
    qpos_pad = jnp.pad(query_pos, pad)
    cpos_pad = jnp.pad(context_pos, pad)
    rff_pad = jnp.pad(rff_B, [(0, 5), (0, 0)])
    b1r = b1.reshape(1, E)
    b2r = b2.reshape(1, E)
    bqr = bq.reshape(1, E)
    bkr = bk.reshape(1, E)
    bvr = bv.reshape(1, E)
    bor = bo.reshape(1, E)

    qp, kc, vc = _encode_project(qpos_pad.reshape(NQT, 8), cpos_pad.reshape(NCT, 8),
                                 rff_pad, W1, b1r, W2, b2r, Wq, bqr, Wk, bkr, Wv, bvr)

    cposT = cpos_pad.transpose(0, 2, 1)
    j, dfull = _knn_chunks(qpos_pad, cposT)

    j_flat = j.reshape(NQT * KNN)
    dflat = dfull.reshape(NQT, NC)

    kg, vg = _sc_refine_gather(j_flat, dflat, kc, vc)

    wseg = jnp.repeat(jnp.eye(NUM_HEADS, dtype=jnp.float32), HD, axis=0)
    out = _attention(qp, kg, vg, wseg, Wo, bor)
    return out.reshape(B, NQ, E), query_pos
